# baseline (speedup 1.0000x reference)
import functools

import jax
import jax.numpy as jnp
from jax import lax
from jax.experimental import pallas as pl
from jax.experimental.pallas import tpu as pltpu
from jax.experimental.pallas import tpu_sc as plsc

_BATCH = 16384
_SUB = 32
_V = 1000000
_RROWS = 250112
_GRID = _RROWS // 128


def _tc_body(x0_ref, x1_ref, x2_ref, x3_ref, o_ref):
    o_ref[...] = jnp.concatenate(
        [x0_ref[...].T, x1_ref[...].T, x2_ref[...].T, x3_ref[...].T], axis=1
    )


_tc_relayout = pl.pallas_call(
    _tc_body,
    grid=(_GRID,),
    in_specs=[
        pl.BlockSpec((_SUB, 128), lambda c, q=q: (0, q * _GRID + c))
        for q in range(4)
    ],
    out_specs=pl.BlockSpec((128, 128), lambda c: (c, 0)),
    out_shape=jax.ShapeDtypeStruct((_RROWS, 128), jnp.float32),
)


def _build_gather(batch):
    info = plsc.get_sparse_core_info()
    nw = info.num_cores * info.num_subcores
    bw = batch // nw
    nchunk = bw // 128
    idx_rows = batch // 128
    mesh = plsc.VectorSubcoreMesh(core_axis_name="c", subcore_axis_name="s")

    @functools.partial(
        pl.kernel,
        mesh=mesh,
        compiler_params=pltpu.CompilerParams(needs_layout_passes=False),
        out_type=jax.ShapeDtypeStruct((batch * 64 // 128, 128), jnp.float32),
        scratch_types=[
            pltpu.VMEM((nchunk, 128), jnp.int32),
            pltpu.VMEM((nchunk, 128), jnp.int32),
            pltpu.VMEM((bw, 128), jnp.float32),
            pltpu.VMEM((bw // 2, 128), jnp.float32),
            pltpu.SemaphoreType.DMA,
        ],
    )
    def gather(k_hbm, q_hbm, r0_hbm, r1_hbm, out_hbm, kv, qv, rows_v, out_v, sem):
        wid = lax.axis_index("s") * info.num_cores + lax.axis_index("c")
        for h, r_hbm in enumerate((r0_hbm, r1_hbm)):
            base_row = h * idx_rows + wid * nchunk
            pltpu.sync_copy(k_hbm.at[pl.ds(base_row, nchunk)], kv)
            pltpu.sync_copy(q_hbm.at[pl.ds(base_row, nchunk)], qv)
            copies = [
                pltpu.async_copy(
                    r_hbm.at[kv.at[j]], rows_v.at[pl.ds(j * 128, 128)], sem
                )
                for j in range(nchunk)
            ]
            for c in copies:
                c.wait()
            def extract(g, _, h=h):
                ivec = jnp.full((16,), g * 16, jnp.int32) + lax.iota(jnp.int32, 16)
                qvec = qv[g // 8, pl.ds((g % 8) * 16, 16)]
                lbase = qvec * 32
                orow = lax.shift_right_logical(ivec, 1)
                obase = (ivec & 1) * 64 + h * 32
                for f in range(_SUB):
                    vals = plsc.load_gather(rows_v, [ivec, lbase + f])
                    plsc.store_scatter(out_v, [orow, obase + f], vals)
                return _

            lax.fori_loop(0, bw // 16, extract, None)
        pltpu.sync_copy(out_v, out_hbm.at[pl.ds(wid * (bw // 2), bw // 2)])

    return gather


_gather = _build_gather(_BATCH)


def kernel(indices, table0, table1):
    idx = indices.astype(jnp.int32)
    kidx = (idx % _RROWS).reshape(2 * _BATCH // 128, 128)
    qidx = (idx // _RROWS).reshape(2 * _BATCH // 128, 128)
    tt0 = table0.T
    tt1 = table1.T
    r0 = _tc_relayout(tt0, tt0, tt0, tt0)
    r1 = _tc_relayout(tt1, tt1, tt1, tt1)
    out = _gather(kidx, qidx, r0, r1)
    return out.reshape(_BATCH, 64)


# speedup vs baseline: 3.5561x; 3.5561x over previous
import functools

import jax
import jax.numpy as jnp
from jax import lax
from jax.experimental import pallas as pl
from jax.experimental.pallas import tpu as pltpu
from jax.experimental.pallas import tpu_sc as plsc

_BATCH = 16384
_SUB = 32
_V = 1000000
_RROWS = 250880
_TC_BLK = 1024
_GRID = _RROWS // _TC_BLK


def _tc_body(x0_ref, x1_ref, x2_ref, x3_ref, o_ref):
    eye = jnp.eye(_SUB, dtype=jnp.float32)
    dn = (((0,), (0,)), ((), ()))
    parts = [
        lax.dot_general(x_ref[...], eye, dn, preferred_element_type=jnp.float32)
        for x_ref in (x0_ref, x1_ref, x2_ref, x3_ref)
    ]
    o_ref[...] = jnp.concatenate(parts, axis=1)


_tc_relayout = pl.pallas_call(
    _tc_body,
    grid=(_GRID,),
    in_specs=[
        pl.BlockSpec(
            (_SUB, _TC_BLK),
            lambda c, q=q: (
                0,
                jnp.minimum(q * _GRID + c, (_V + _TC_BLK - 1) // _TC_BLK - 1),
            ),
        )
        for q in range(4)
    ],
    out_specs=pl.BlockSpec((_TC_BLK, 128), lambda c: (c, 0)),
    out_shape=jax.ShapeDtypeStruct((_RROWS, 128), jnp.float32),
)


def _build_gather(batch):
    info = plsc.get_sparse_core_info()
    nw = info.num_cores * info.num_subcores
    bw = batch // nw
    nchunk = bw // 128
    idx_rows = batch // 128
    mesh = plsc.VectorSubcoreMesh(core_axis_name="c", subcore_axis_name="s")

    @functools.partial(
        pl.kernel,
        mesh=mesh,
        compiler_params=pltpu.CompilerParams(needs_layout_passes=False),
        out_type=jax.ShapeDtypeStruct((batch * 64 // 128, 128), jnp.float32),
        scratch_types=[
            pltpu.VMEM((nchunk, 128), jnp.int32),
            pltpu.VMEM((nchunk, 128), jnp.int32),
            pltpu.VMEM((bw, 128), jnp.float32),
            pltpu.VMEM((bw // 2, 128), jnp.float32),
            pltpu.SemaphoreType.DMA,
        ],
    )
    def gather(k_hbm, q_hbm, r0_hbm, r1_hbm, out_hbm, kv, qv, rows_v, out_v, sem):
        wid = lax.axis_index("s") * info.num_cores + lax.axis_index("c")
        for h, r_hbm in enumerate((r0_hbm, r1_hbm)):
            base_row = h * idx_rows + wid * nchunk
            pltpu.sync_copy(k_hbm.at[pl.ds(base_row, nchunk)], kv)
            pltpu.sync_copy(q_hbm.at[pl.ds(base_row, nchunk)], qv)
            copies = [
                pltpu.async_copy(
                    r_hbm.at[kv.at[j]], rows_v.at[pl.ds(j * 128, 128)], sem
                )
                for j in range(nchunk)
            ]
            for c in copies:
                c.wait()
            def extract(g, _, h=h):
                ivec = jnp.full((16,), g * 16, jnp.int32) + lax.iota(jnp.int32, 16)
                qvec = qv[g // 8, pl.ds((g % 8) * 16, 16)]
                lbase = qvec * 32
                orow = lax.shift_right_logical(ivec, 1)
                obase = (ivec & 1) * 64 + h * 32
                for f in range(_SUB):
                    vals = plsc.load_gather(rows_v, [ivec, lbase + f])
                    plsc.store_scatter(out_v, [orow, obase + f], vals)
                return _

            lax.fori_loop(0, bw // 16, extract, None)
        pltpu.sync_copy(out_v, out_hbm.at[pl.ds(wid * (bw // 2), bw // 2)])

    return gather


_gather = _build_gather(_BATCH)


def kernel(indices, table0, table1):
    idx = indices.astype(jnp.int32)
    kidx = (idx % _RROWS).reshape(2 * _BATCH // 128, 128)
    qidx = (idx // _RROWS).reshape(2 * _BATCH // 128, 128)
    tt0 = table0.T
    tt1 = table1.T
    r0 = _tc_relayout(tt0, tt0, tt0, tt0)
    r1 = _tc_relayout(tt1, tt1, tt1, tt1)
    out = _gather(kidx, qidx, r0, r1)
    return out.reshape(_BATCH, 64)


# speedup vs baseline: 3.9635x; 1.1146x over previous
import functools

import jax
import jax.numpy as jnp
from jax import lax
from jax.experimental import pallas as pl
from jax.experimental.pallas import tpu as pltpu
from jax.experimental.pallas import tpu_sc as plsc

_BATCH = 16384
_SUB = 32
_V = 1000000
_RROWS = 250880
_TC_BLK = 1024
_GRID = _RROWS // _TC_BLK


def _tc_body(x0_ref, x1_ref, x2_ref, x3_ref, o_ref):
    eye = jnp.eye(_SUB, dtype=jnp.bfloat16)
    dn = (((0,), (0,)), ((), ()))
    parts = [
        lax.dot_general(
            x_ref[...].astype(jnp.bfloat16),
            eye,
            dn,
            preferred_element_type=jnp.float32,
        )
        for x_ref in (x0_ref, x1_ref, x2_ref, x3_ref)
    ]
    o_ref[...] = jnp.concatenate(parts, axis=1)


_tc_relayout = pl.pallas_call(
    _tc_body,
    grid=(_GRID,),
    in_specs=[
        pl.BlockSpec(
            (_SUB, _TC_BLK),
            lambda c, q=q: (
                0,
                jnp.minimum(q * _GRID + c, (_V + _TC_BLK - 1) // _TC_BLK - 1),
            ),
        )
        for q in range(4)
    ],
    out_specs=pl.BlockSpec((_TC_BLK, 128), lambda c: (c, 0)),
    out_shape=jax.ShapeDtypeStruct((_RROWS, 128), jnp.float32),
)


def _build_gather(batch):
    info = plsc.get_sparse_core_info()
    nw = info.num_cores * info.num_subcores
    bw = batch // nw
    nchunk = bw // 128
    idx_rows = batch // 128
    mesh = plsc.VectorSubcoreMesh(core_axis_name="c", subcore_axis_name="s")

    @functools.partial(
        pl.kernel,
        mesh=mesh,
        compiler_params=pltpu.CompilerParams(needs_layout_passes=False),
        out_type=jax.ShapeDtypeStruct((batch * 64 // 128, 128), jnp.float32),
        scratch_types=[
            pltpu.VMEM((nchunk, 128), jnp.int32),
            pltpu.VMEM((nchunk, 128), jnp.int32),
            pltpu.VMEM((bw, 128), jnp.float32),
            pltpu.VMEM((bw // 2, 128), jnp.float32),
            pltpu.SemaphoreType.DMA,
        ],
    )
    def gather(k_hbm, q_hbm, r0_hbm, r1_hbm, out_hbm, kv, qv, rows_v, out_v, sem):
        wid = lax.axis_index("s") * info.num_cores + lax.axis_index("c")
        for h, r_hbm in enumerate((r0_hbm, r1_hbm)):
            base_row = h * idx_rows + wid * nchunk
            pltpu.sync_copy(k_hbm.at[pl.ds(base_row, nchunk)], kv)
            pltpu.sync_copy(q_hbm.at[pl.ds(base_row, nchunk)], qv)
            copies = [
                pltpu.async_copy(
                    r_hbm.at[kv.at[j]], rows_v.at[pl.ds(j * 128, 128)], sem
                )
                for j in range(nchunk)
            ]
            for c in copies:
                c.wait()
            def extract(g, _, h=h):
                ivec = jnp.full((16,), g * 16, jnp.int32) + lax.iota(jnp.int32, 16)
                qvec = qv[g // 8, pl.ds((g % 8) * 16, 16)]
                lbase = qvec * 32
                orow = lax.shift_right_logical(ivec, 1)
                obase = (ivec & 1) * 64 + h * 32
                for f in range(_SUB):
                    vals = plsc.load_gather(rows_v, [ivec, lbase + f])
                    plsc.store_scatter(out_v, [orow, obase + f], vals)
                return _

            lax.fori_loop(0, bw // 16, extract, None)
        pltpu.sync_copy(out_v, out_hbm.at[pl.ds(wid * (bw // 2), bw // 2)])

    return gather


_gather = _build_gather(_BATCH)


def kernel(indices, table0, table1):
    idx = indices.astype(jnp.int32)
    kidx = (idx % _RROWS).reshape(2 * _BATCH // 128, 128)
    qidx = (idx // _RROWS).reshape(2 * _BATCH // 128, 128)
    tt0 = table0.T
    tt1 = table1.T
    r0 = _tc_relayout(tt0, tt0, tt0, tt0)
    r1 = _tc_relayout(tt1, tt1, tt1, tt1)
    out = _gather(kidx, qidx, r0, r1)
    return out.reshape(_BATCH, 64)


# speedup vs baseline: 5.4405x; 1.3726x over previous
import functools

import jax
import jax.numpy as jnp
from jax import lax
from jax.experimental import pallas as pl
from jax.experimental.pallas import tpu as pltpu
from jax.experimental.pallas import tpu_sc as plsc

_BATCH = 16384
_SUB = 32
_V = 1000000
_RROWS = 253952
_TC_BLK = 8192
_GRID = _RROWS // _TC_BLK


def _tc_body(x0_ref, x1_ref, x2_ref, x3_ref, o_ref):
    eye = jnp.eye(_SUB, dtype=jnp.bfloat16)
    dn = (((0,), (0,)), ((), ()))
    parts = [
        lax.dot_general(
            x_ref[...].astype(jnp.bfloat16),
            eye,
            dn,
            preferred_element_type=jnp.float32,
        )
        for x_ref in (x0_ref, x1_ref, x2_ref, x3_ref)
    ]
    o_ref[...] = jnp.concatenate(parts, axis=1)


_tc_relayout = pl.pallas_call(
    _tc_body,
    grid=(_GRID,),
    in_specs=[
        pl.BlockSpec(
            (_SUB, _TC_BLK),
            lambda c, q=q: (
                0,
                jnp.minimum(q * _GRID + c, (_V + _TC_BLK - 1) // _TC_BLK - 1),
            ),
        )
        for q in range(4)
    ],
    out_specs=pl.BlockSpec((_TC_BLK, 128), lambda c: (c, 0)),
    out_shape=jax.ShapeDtypeStruct((_RROWS, 128), jnp.float32),
)


def _build_gather(batch):
    info = plsc.get_sparse_core_info()
    nw = info.num_cores * info.num_subcores
    bw = batch // nw
    nchunk = bw // 128
    idx_rows = batch // 128
    mesh = plsc.VectorSubcoreMesh(core_axis_name="c", subcore_axis_name="s")

    @functools.partial(
        pl.kernel,
        mesh=mesh,
        compiler_params=pltpu.CompilerParams(needs_layout_passes=False),
        out_type=jax.ShapeDtypeStruct((batch * 64 // 128, 128), jnp.float32),
        scratch_types=[
            pltpu.VMEM((nchunk, 128), jnp.int32),
            pltpu.VMEM((nchunk, 128), jnp.int32),
            pltpu.VMEM((bw, 128), jnp.float32),
            pltpu.VMEM((bw // 2, 128), jnp.float32),
            pltpu.SemaphoreType.DMA,
        ],
    )
    def gather(k_hbm, q_hbm, r0_hbm, r1_hbm, out_hbm, kv, qv, rows_v, out_v, sem):
        wid = lax.axis_index("s") * info.num_cores + lax.axis_index("c")
        for h, r_hbm in enumerate((r0_hbm, r1_hbm)):
            base_row = h * idx_rows + wid * nchunk
            pltpu.sync_copy(k_hbm.at[pl.ds(base_row, nchunk)], kv)
            pltpu.sync_copy(q_hbm.at[pl.ds(base_row, nchunk)], qv)
            copies = [
                pltpu.async_copy(
                    r_hbm.at[kv.at[j]], rows_v.at[pl.ds(j * 128, 128)], sem
                )
                for j in range(nchunk)
            ]
            for c in copies:
                c.wait()
            def extract(g, _, h=h):
                ivec = jnp.full((16,), g * 16, jnp.int32) + lax.iota(jnp.int32, 16)
                qvec = qv[g // 8, pl.ds((g % 8) * 16, 16)]
                lbase = qvec * 32
                orow = lax.shift_right_logical(ivec, 1)
                obase = (ivec & 1) * 64 + h * 32
                for f in range(_SUB):
                    vals = plsc.load_gather(rows_v, [ivec, lbase + f])
                    plsc.store_scatter(out_v, [orow, obase + f], vals)
                return _

            lax.fori_loop(0, bw // 16, extract, None)
        pltpu.sync_copy(out_v, out_hbm.at[pl.ds(wid * (bw // 2), bw // 2)])

    return gather


_gather = _build_gather(_BATCH)


def kernel(indices, table0, table1):
    idx = indices.astype(jnp.int32)
    kidx = (idx % _RROWS).reshape(2 * _BATCH // 128, 128)
    qidx = (idx // _RROWS).reshape(2 * _BATCH // 128, 128)
    tt0 = table0.T
    tt1 = table1.T
    r0 = _tc_relayout(tt0, tt0, tt0, tt0)
    r1 = _tc_relayout(tt1, tt1, tt1, tt1)
    out = _gather(kidx, qidx, r0, r1)
    return out.reshape(_BATCH, 64)
